# Initial kernel scaffold; baseline (speedup 1.0000x reference)
import jax
import jax.numpy as jnp
from jax.experimental import pallas as pl


def kernel(x, token_table, pos_table):
    raise NotImplementedError("write your pallas kernel here")


# speedup vs baseline: 3.0986x; 3.0986x over previous
import functools

import jax
import jax.numpy as jnp
from jax import lax
from jax.experimental import pallas as pl
from jax.experimental.pallas import tpu as pltpu
from jax.experimental.pallas import tpu_sc as plsc

_MAXLEN = 200
_EMBED = 64
_BATCH = 4096
_NC = 2
_NS = 16
_NW = _NC * _NS
_SEQ_PER_W = _BATCH // _NW
_HALF = _MAXLEN // 2


def _make_sc_kernel():
    mesh = plsc.VectorSubcoreMesh(core_axis_name="c", subcore_axis_name="s")

    @functools.partial(
        pl.kernel,
        mesh=mesh,
        compiler_params=pltpu.CompilerParams(use_tc_tiling_on_sc=False),
        out_type=jax.ShapeDtypeStruct((_BATCH * _MAXLEN, _EMBED), jnp.float32),
        scratch_types=[
            pltpu.VMEM((_MAXLEN, _EMBED), jnp.float32),
            pltpu.VMEM((2, _HALF), jnp.int32),
            pltpu.VMEM((_MAXLEN, _EMBED), jnp.float32),
            pltpu.SemaphoreType.DMA,
        ],
    )
    def k(x_hbm, tok_hbm, pos_hbm, out_hbm, pos_v, idx_v, rows_v, sem):
        wid = lax.axis_index("s") * _NC + lax.axis_index("c")
        seq0 = wid * _SEQ_PER_W
        pltpu.sync_copy(pos_hbm, pos_v)

        def body(i, carry):
            seq = seq0 + i
            row0 = seq * _MAXLEN
            pltpu.sync_copy(x_hbm.at[pl.ds(seq * 2, 2)], idx_v)
            cp0 = pltpu.async_copy(
                tok_hbm.at[idx_v.at[0]], rows_v.at[pl.ds(0, _HALF)], sem)
            cp1 = pltpu.async_copy(
                tok_hbm.at[idx_v.at[1]], rows_v.at[pl.ds(_HALF, _HALF)], sem)
            cp0.wait()
            cp1.wait()

            def radd(r, c2):
                for c in range(_EMBED // 16):
                    sl = pl.ds(c * 16, 16)
                    rows_v[r, sl] = rows_v[r, sl] + pos_v[r, sl]
                return c2
            lax.fori_loop(0, _MAXLEN, radd, 0)
            pltpu.sync_copy(rows_v, out_hbm.at[pl.ds(row0, _MAXLEN)])
            return carry

        lax.fori_loop(0, _SEQ_PER_W, body, 0)

    return k


def kernel(x, token_table, pos_table):
    x_rows = x.astype(jnp.int32).reshape(_BATCH * _MAXLEN // _HALF, _HALF)
    out = _make_sc_kernel()(x_rows, token_table, pos_table)
    return out.reshape(_BATCH, _MAXLEN, _EMBED)


# speedup vs baseline: 4.2271x; 1.3642x over previous
import functools

import jax
import jax.numpy as jnp
from jax import lax
from jax.experimental import pallas as pl
from jax.experimental.pallas import tpu as pltpu
from jax.experimental.pallas import tpu_sc as plsc

_MAXLEN = 200
_EMBED = 64
_BATCH = 4096
_NC = 2
_NS = 16
_NW = _NC * _NS
_SEQ_PER_W = _BATCH // _NW
_HALF = _MAXLEN // 2
_NBUF = 4


def _make_sc_kernel():
    mesh = plsc.VectorSubcoreMesh(core_axis_name="c", subcore_axis_name="s")

    @functools.partial(
        pl.kernel,
        mesh=mesh,
        compiler_params=pltpu.CompilerParams(use_tc_tiling_on_sc=False),
        out_type=jax.ShapeDtypeStruct((_BATCH * _MAXLEN, _EMBED), jnp.float32),
        scratch_types=[
            pltpu.VMEM_SHARED((_MAXLEN, _EMBED), jnp.float32),
            pltpu.VMEM((2 * _SEQ_PER_W, _HALF), jnp.int32),
            pltpu.VMEM((_NBUF, _MAXLEN, _EMBED), jnp.float32),
        ]
        + [pltpu.SemaphoreType.DMA] * (3 * _NBUF),
    )
    def k(x_hbm, tok_hbm, pos_hbm, out_hbm, pos_sh, idx_v, rows_v, *sems):
        isems, gsems, osems = sems[:_NBUF], sems[_NBUF:2 * _NBUF], sems[2 * _NBUF:]
        sid = lax.axis_index("s")
        wid = sid * _NC + lax.axis_index("c")
        seq0 = wid * _SEQ_PER_W

        @pl.when(sid == 0)
        def _():
            pltpu.sync_copy(pos_hbm, pos_sh)
        plsc.subcore_barrier()
        pltpu.sync_copy(x_hbm.at[pl.ds(seq0 * 2, 2 * _SEQ_PER_W)], idx_v)

        def fire_init(b):
            pltpu.async_copy(pos_sh, rows_v.at[b], isems[b])

        def wait_init(b):
            pltpu.make_async_copy(pos_sh, rows_v.at[b], isems[b]).wait()

        def fire_gather(i, b):
            pltpu.async_copy(tok_hbm.at[idx_v.at[2 * i]],
                             rows_v.at[b, pl.ds(0, _HALF)], gsems[b], add=True)
            pltpu.async_copy(tok_hbm.at[idx_v.at[2 * i + 1]],
                             rows_v.at[b, pl.ds(_HALF, _HALF)], gsems[b], add=True)

        def wait_gather(b):
            pltpu.make_async_copy(tok_hbm.at[idx_v.at[0]],
                                  rows_v.at[b, pl.ds(0, _HALF)], gsems[b]).wait()
            pltpu.make_async_copy(tok_hbm.at[idx_v.at[0]],
                                  rows_v.at[b, pl.ds(_HALF, _HALF)], gsems[b]).wait()

        def fire_store(i, b):
            row0 = (seq0 + i) * _MAXLEN
            pltpu.async_copy(rows_v.at[b], out_hbm.at[pl.ds(row0, _MAXLEN)],
                             osems[b])

        def wait_store(i, b):
            row0 = (seq0 + i) * _MAXLEN
            pltpu.make_async_copy(rows_v.at[b],
                                  out_hbm.at[pl.ds(row0, _MAXLEN)],
                                  osems[b]).wait()

        for b in range(_NBUF):
            fire_init(b)
        for b in range(_NBUF):
            wait_init(b)
            fire_gather(b, b)

        def outer(g, carry):
            i0 = g * _NBUF
            for b in range(_NBUF):
                wait_gather(b)
                fire_store(i0 + b, b)
            for b in range(_NBUF):
                wait_store(i0 + b, b)
                fire_init(b)
            for b in range(_NBUF):
                wait_init(b)
                fire_gather(i0 + _NBUF + b, b)
            return carry

        lax.fori_loop(0, _SEQ_PER_W // _NBUF - 1, outer, 0)

        i0 = _SEQ_PER_W - _NBUF
        for b in range(_NBUF):
            wait_gather(b)
            fire_store(i0 + b, b)
        for b in range(_NBUF):
            wait_store(i0 + b, b)

    return k


def kernel(x, token_table, pos_table):
    x_rows = x.astype(jnp.int32).reshape(_BATCH * _MAXLEN // _HALF, _HALF)
    out = _make_sc_kernel()(x_rows, token_table, pos_table)
    return out.reshape(_BATCH, _MAXLEN, _EMBED)


# speedup vs baseline: 4.2300x; 1.0007x over previous
import functools

import jax
import jax.numpy as jnp
from jax import lax
from jax.experimental import pallas as pl
from jax.experimental.pallas import tpu as pltpu
from jax.experimental.pallas import tpu_sc as plsc

_MAXLEN = 200
_EMBED = 64
_BATCH = 4096
_NC = 2
_NS = 16
_NW = _NC * _NS
_SEQ_PER_W = _BATCH // _NW
_CHUNK_A = 128
_CHUNK_B = _MAXLEN - _CHUNK_A
_NBUF = 4


def _make_sc_kernel():
    mesh = plsc.VectorSubcoreMesh(core_axis_name="c", subcore_axis_name="s")

    @functools.partial(
        pl.kernel,
        mesh=mesh,
        compiler_params=pltpu.CompilerParams(use_tc_tiling_on_sc=False),
        out_type=jax.ShapeDtypeStruct((_BATCH, _MAXLEN, _EMBED), jnp.float32),
        scratch_types=[
            pltpu.VMEM_SHARED((_MAXLEN, _EMBED), jnp.float32),
            pltpu.VMEM((_SEQ_PER_W, _MAXLEN), jnp.int32),
            pltpu.VMEM((_NBUF, _MAXLEN, _EMBED), jnp.float32),
        ]
        + [pltpu.SemaphoreType.DMA] * (3 * _NBUF),
    )
    def k(x_hbm, tok_hbm, pos_hbm, out_hbm, pos_sh, idx_v, rows_v, *sems):
        isems, gsems, osems = sems[:_NBUF], sems[_NBUF:2 * _NBUF], sems[2 * _NBUF:]
        sid = lax.axis_index("s")
        wid = sid * _NC + lax.axis_index("c")
        seq0 = wid * _SEQ_PER_W

        @pl.when(sid == 0)
        def _():
            pltpu.sync_copy(pos_hbm, pos_sh)
        plsc.subcore_barrier()
        pltpu.sync_copy(x_hbm.at[pl.ds(seq0, _SEQ_PER_W)], idx_v)

        def fire_init(b):
            pltpu.async_copy(pos_sh, rows_v.at[b], isems[b])

        def wait_init(b):
            pltpu.make_async_copy(pos_sh, rows_v.at[b], isems[b]).wait()

        def fire_gather(i, b):
            pltpu.async_copy(tok_hbm.at[idx_v.at[i, pl.ds(0, _CHUNK_A)]],
                             rows_v.at[b, pl.ds(0, _CHUNK_A)], gsems[b],
                             add=True)
            pltpu.async_copy(tok_hbm.at[idx_v.at[i, pl.ds(_CHUNK_A, _CHUNK_B)]],
                             rows_v.at[b, pl.ds(_CHUNK_A, _CHUNK_B)], gsems[b],
                             add=True)

        def wait_gather(b):
            pltpu.make_async_copy(tok_hbm.at[idx_v.at[0, pl.ds(0, _CHUNK_A)]],
                                  rows_v.at[b, pl.ds(0, _CHUNK_A)],
                                  gsems[b]).wait()
            pltpu.make_async_copy(tok_hbm.at[idx_v.at[0, pl.ds(0, _CHUNK_B)]],
                                  rows_v.at[b, pl.ds(_CHUNK_A, _CHUNK_B)],
                                  gsems[b]).wait()

        def fire_store(i, b):
            pltpu.async_copy(rows_v.at[b], out_hbm.at[seq0 + i], osems[b])

        def wait_store(i, b):
            pltpu.make_async_copy(rows_v.at[b], out_hbm.at[seq0 + i],
                                  osems[b]).wait()

        for b in range(_NBUF):
            fire_init(b)
        for b in range(_NBUF):
            wait_init(b)
            fire_gather(b, b)

        def outer(g, carry):
            i0 = g * _NBUF
            for b in range(_NBUF):
                wait_gather(b)
                fire_store(i0 + b, b)
            for b in range(_NBUF):
                wait_store(i0 + b, b)
                fire_init(b)
            for b in range(_NBUF):
                wait_init(b)
                fire_gather(i0 + _NBUF + b, b)
            return carry

        lax.fori_loop(0, _SEQ_PER_W // _NBUF - 1, outer, 0)

        i0 = _SEQ_PER_W - _NBUF
        for b in range(_NBUF):
            wait_gather(b)
            fire_store(i0 + b, b)
        for b in range(_NBUF):
            wait_store(i0 + b, b)

    return k


def kernel(x, token_table, pos_table):
    return _make_sc_kernel()(x.astype(jnp.int32), token_table, pos_table)


# speedup vs baseline: 4.2355x; 1.0013x over previous
import functools

import jax
import jax.numpy as jnp
from jax import lax
from jax.experimental import pallas as pl
from jax.experimental.pallas import tpu as pltpu
from jax.experimental.pallas import tpu_sc as plsc

_MAXLEN = 200
_EMBED = 64
_BATCH = 4096
_NC = 2
_NS = 16
_NW = _NC * _NS
_SEQ_PER_W = _BATCH // _NW
_CHUNK_A = 128
_CHUNK_B = _MAXLEN - _CHUNK_A
_NBUF = 4


def _make_sc_kernel():
    mesh = plsc.VectorSubcoreMesh(core_axis_name="c", subcore_axis_name="s")

    @functools.partial(
        pl.kernel,
        mesh=mesh,
        compiler_params=pltpu.CompilerParams(use_tc_tiling_on_sc=False),
        out_type=jax.ShapeDtypeStruct((_BATCH * _MAXLEN, _EMBED), jnp.float32),
        scratch_types=[
            pltpu.VMEM_SHARED((_MAXLEN, _EMBED), jnp.float32),
            pltpu.VMEM((_SEQ_PER_W, _MAXLEN), jnp.int32),
            pltpu.VMEM((_NBUF, _MAXLEN, _EMBED), jnp.float32),
        ]
        + [pltpu.SemaphoreType.DMA] * (3 * _NBUF),
    )
    def k(x_hbm, tok_hbm, pos_hbm, out_hbm, pos_sh, idx_v, rows_v, *sems):
        isems, gsems, osems = sems[:_NBUF], sems[_NBUF:2 * _NBUF], sems[2 * _NBUF:]
        sid = lax.axis_index("s")
        wid = sid * _NC + lax.axis_index("c")
        seq0 = wid * _SEQ_PER_W

        @pl.when(sid == 0)
        def _():
            pltpu.sync_copy(pos_hbm, pos_sh)
        plsc.subcore_barrier()
        pltpu.sync_copy(x_hbm.at[pl.ds(seq0, _SEQ_PER_W)], idx_v)

        def fire_init(b):
            pltpu.async_copy(pos_sh, rows_v.at[b], isems[b])

        def wait_init(b):
            pltpu.make_async_copy(pos_sh, rows_v.at[b], isems[b]).wait()

        def fire_gather(i, b):
            pltpu.async_copy(tok_hbm.at[idx_v.at[i, pl.ds(0, _CHUNK_A)]],
                             rows_v.at[b, pl.ds(0, _CHUNK_A)], gsems[b],
                             add=True)
            pltpu.async_copy(tok_hbm.at[idx_v.at[i, pl.ds(_CHUNK_A, _CHUNK_B)]],
                             rows_v.at[b, pl.ds(_CHUNK_A, _CHUNK_B)], gsems[b],
                             add=True)

        def wait_gather(b):
            pltpu.make_async_copy(tok_hbm.at[idx_v.at[0, pl.ds(0, _CHUNK_A)]],
                                  rows_v.at[b, pl.ds(0, _CHUNK_A)],
                                  gsems[b]).wait()
            pltpu.make_async_copy(tok_hbm.at[idx_v.at[0, pl.ds(0, _CHUNK_B)]],
                                  rows_v.at[b, pl.ds(_CHUNK_A, _CHUNK_B)],
                                  gsems[b]).wait()

        def fire_store(i, b):
            pltpu.async_copy(rows_v.at[b],
                             out_hbm.at[pl.ds((seq0 + i) * _MAXLEN, _MAXLEN)],
                             osems[b])

        def wait_store(i, b):
            pltpu.make_async_copy(rows_v.at[b],
                                  out_hbm.at[pl.ds((seq0 + i) * _MAXLEN, _MAXLEN)],
                                  osems[b]).wait()

        for b in range(_NBUF):
            fire_init(b)
        for b in range(_NBUF):
            wait_init(b)
            fire_gather(b, b)

        def outer(g, carry):
            i0 = g * _NBUF
            for b in range(_NBUF):
                wait_gather(b)
                fire_store(i0 + b, b)
            for b in range(_NBUF):
                wait_store(i0 + b, b)
                fire_init(b)
            for b in range(_NBUF):
                wait_init(b)
                fire_gather(i0 + _NBUF + b, b)
            return carry

        lax.fori_loop(0, _SEQ_PER_W // _NBUF - 1, outer, 0)

        i0 = _SEQ_PER_W - _NBUF
        for b in range(_NBUF):
            wait_gather(b)
            fire_store(i0 + b, b)
        for b in range(_NBUF):
            wait_store(i0 + b, b)

    return k


def kernel(x, token_table, pos_table):
    out = _make_sc_kernel()(x.astype(jnp.int32), token_table, pos_table)
    return out.reshape(_BATCH, _MAXLEN, _EMBED)


# speedup vs baseline: 4.2373x; 1.0004x over previous
import functools

import jax
import jax.numpy as jnp
from jax import lax
from jax.experimental import pallas as pl
from jax.experimental.pallas import tpu as pltpu
from jax.experimental.pallas import tpu_sc as plsc

_MAXLEN = 200
_EMBED = 64
_BATCH = 4096
_NC = 2
_NS = 16
_NW = _NC * _NS
_SEQ_PER_W = _BATCH // _NW
_CHUNK_A = 128
_CHUNK_B = _MAXLEN - _CHUNK_A
_NBUF = 8


def _make_sc_kernel():
    mesh = plsc.VectorSubcoreMesh(core_axis_name="c", subcore_axis_name="s")

    @functools.partial(
        pl.kernel,
        mesh=mesh,
        compiler_params=pltpu.CompilerParams(use_tc_tiling_on_sc=False),
        out_type=jax.ShapeDtypeStruct((_BATCH * _MAXLEN, _EMBED), jnp.float32),
        scratch_types=[
            pltpu.VMEM_SHARED((_MAXLEN, _EMBED), jnp.float32),
            pltpu.VMEM((_SEQ_PER_W, _MAXLEN), jnp.int32),
            pltpu.VMEM((_NBUF, _MAXLEN, _EMBED), jnp.float32),
        ]
        + [pltpu.SemaphoreType.DMA] * (3 * _NBUF),
    )
    def k(x_hbm, tok_hbm, pos_hbm, out_hbm, pos_sh, idx_v, rows_v, *sems):
        isems, gsems, osems = sems[:_NBUF], sems[_NBUF:2 * _NBUF], sems[2 * _NBUF:]
        sid = lax.axis_index("s")
        wid = sid * _NC + lax.axis_index("c")
        seq0 = wid * _SEQ_PER_W

        @pl.when(sid == 0)
        def _():
            pltpu.sync_copy(pos_hbm, pos_sh)
        plsc.subcore_barrier()
        pltpu.sync_copy(x_hbm.at[pl.ds(seq0, _SEQ_PER_W)], idx_v)

        def fire_init(b):
            pltpu.async_copy(pos_sh, rows_v.at[b], isems[b])

        def wait_init(b):
            pltpu.make_async_copy(pos_sh, rows_v.at[b], isems[b]).wait()

        def fire_gather(i, b):
            pltpu.async_copy(tok_hbm.at[idx_v.at[i, pl.ds(0, _CHUNK_A)]],
                             rows_v.at[b, pl.ds(0, _CHUNK_A)], gsems[b],
                             add=True)
            pltpu.async_copy(tok_hbm.at[idx_v.at[i, pl.ds(_CHUNK_A, _CHUNK_B)]],
                             rows_v.at[b, pl.ds(_CHUNK_A, _CHUNK_B)], gsems[b],
                             add=True)

        def wait_gather(b):
            pltpu.make_async_copy(tok_hbm.at[idx_v.at[0, pl.ds(0, _CHUNK_A)]],
                                  rows_v.at[b, pl.ds(0, _CHUNK_A)],
                                  gsems[b]).wait()
            pltpu.make_async_copy(tok_hbm.at[idx_v.at[0, pl.ds(0, _CHUNK_B)]],
                                  rows_v.at[b, pl.ds(_CHUNK_A, _CHUNK_B)],
                                  gsems[b]).wait()

        def fire_store(i, b):
            pltpu.async_copy(rows_v.at[b],
                             out_hbm.at[pl.ds((seq0 + i) * _MAXLEN, _MAXLEN)],
                             osems[b])

        def wait_store(i, b):
            pltpu.make_async_copy(rows_v.at[b],
                                  out_hbm.at[pl.ds((seq0 + i) * _MAXLEN, _MAXLEN)],
                                  osems[b]).wait()

        for b in range(_NBUF):
            fire_init(b)
        for b in range(_NBUF):
            wait_init(b)
            fire_gather(b, b)

        def outer(g, carry):
            i0 = g * _NBUF
            for b in range(_NBUF):
                wait_gather(b)
                fire_store(i0 + b, b)
            for b in range(_NBUF):
                wait_store(i0 + b, b)
                fire_init(b)
            for b in range(_NBUF):
                wait_init(b)
                fire_gather(i0 + _NBUF + b, b)
            return carry

        lax.fori_loop(0, _SEQ_PER_W // _NBUF - 1, outer, 0)

        i0 = _SEQ_PER_W - _NBUF
        for b in range(_NBUF):
            wait_gather(b)
            fire_store(i0 + b, b)
        for b in range(_NBUF):
            wait_store(i0 + b, b)

    return k


def kernel(x, token_table, pos_table):
    out = _make_sc_kernel()(x.astype(jnp.int32), token_table, pos_table)
    return out.reshape(_BATCH, _MAXLEN, _EMBED)
